# baseline (speedup 1.0000x reference)
import functools

import jax
import jax.numpy as jnp
from jax import lax
from jax.experimental import pallas as pl
from jax.experimental.pallas import tpu as pltpu
from jax.experimental.pallas import tpu_sc as plsc

TOKENS_DIM = 100000
EMBEDDING_DIM = 64
BATCH = 16384
HIST_LEN = 50
_B = BATCH * HIST_LEN

_info = plsc.get_sparse_core_info()
_NC = _info.num_cores
_NS = _info.num_subcores
_NW = _NC * _NS
_BH = BATCH // 128
_JPW = _BH // _NW

_mesh = plsc.VectorSubcoreMesh(core_axis_name="c", subcore_axis_name="s")


@functools.partial(
    pl.kernel,
    mesh=_mesh,
    compiler_params=pltpu.CompilerParams(
        use_tc_tiling_on_sc=False, needs_layout_passes=False
    ),
    out_type=jax.ShapeDtypeStruct(
        (HIST_LEN, 8, _BH, 8, 128), jnp.float32
    ),
    scratch_types=[
        pltpu.VMEM((HIST_LEN, 128 * _JPW), jnp.int32),
        pltpu.VMEM((128, EMBEDDING_DIM), jnp.float32),
        pltpu.VMEM((128, EMBEDDING_DIM), jnp.float32),
        pltpu.VMEM((128, EMBEDDING_DIM), jnp.float32),
        pltpu.VMEM((128, EMBEDDING_DIM), jnp.float32),
        pltpu.VMEM((8, 8, 128), jnp.float32),
        pltpu.VMEM((8, 8, 128), jnp.float32),
        pltpu.VMEM((8, 8, 128), jnp.float32),
        pltpu.VMEM((8, 8, 128), jnp.float32),
        pltpu.SemaphoreType.DMA,
        pltpu.SemaphoreType.DMA,
        pltpu.SemaphoreType.DMA,
        pltpu.SemaphoreType.DMA,
        pltpu.SemaphoreType.DMA,
        pltpu.SemaphoreType.DMA,
        pltpu.SemaphoreType.DMA,
        pltpu.SemaphoreType.DMA,
    ],
)
def _gather_kernel(idxt_hbm, table_hbm, out_hbm, idx_v,
                   r0, r1, r2, r3, tb0, tb1, tb2, tb3,
                   g0, g1, g2, g3, s0, s1, s2, s3):
    wid = lax.axis_index("s") * _NC + lax.axis_index("c")
    rows = (r0, r1, r2, r3)
    tbuf = (tb0, tb1, tb2, tb3)
    gsem = (g0, g1, g2, g3)
    ssem = (s0, s1, s2, s3)

    pltpu.sync_copy(idxt_hbm.at[:, pl.ds(wid * 128 * _JPW, 128 * _JPW)], idx_v)

    iota = lax.iota(jnp.int32, 16)
    iotak = [iota + 16 * k for k in range(8)]

    def gather_desc(t, j):
        return pltpu.make_async_copy(
            table_hbm.at[idx_v.at[t, pl.ds(j * 128, 128)]],
            rows[j],
            gsem[j],
        )

    def store_desc(t, j):
        return pltpu.make_async_copy(
            tbuf[j],
            out_hbm.at[t, :, wid * _JPW + j],
            ssem[j],
        )

    for j in range(_JPW):
        gather_desc(0, j).start()
        store_desc(0, j).start()

    def body(t, carry):
        for j in range(_JPW):
            gather_desc(t, j).wait()
            store_desc(t, j).wait()
            for d in range(EMBEDDING_DIM):
                dvec = jnp.full((16,), d, dtype=jnp.int32)
                for k in range(8):
                    v = plsc.load_gather(rows[j], [iotak[k], dvec])
                    tbuf[j][d // 8, d % 8, pl.ds(16 * k, 16)] = v
            @pl.when(t < HIST_LEN - 1)
            def _():
                gather_desc(t + 1, j).start()
            store_desc(t, j).start()
        return carry

    lax.fori_loop(0, HIST_LEN, body, 0)

    for j in range(_JPW):
        store_desc(HIST_LEN - 1, j).wait()


def kernel(indices, table):
    idxt = indices.T.astype(jnp.int32)
    out5 = _gather_kernel(idxt, table)
    return out5.transpose(2, 4, 0, 1, 3).reshape(BATCH, HIST_LEN, EMBEDDING_DIM)


# speedup vs baseline: 2.5644x; 2.5644x over previous
import functools

import jax
import jax.numpy as jnp
from jax import lax
from jax.experimental import pallas as pl
from jax.experimental.pallas import tpu as pltpu
from jax.experimental.pallas import tpu_sc as plsc

TOKENS_DIM = 100000
EMBEDDING_DIM = 64
BATCH = 16384
HIST_LEN = 50
_B = BATCH * HIST_LEN

_info = plsc.get_sparse_core_info()
_NC = _info.num_cores
_NS = _info.num_subcores
_NW = _NC * _NS
_BH = BATCH // 128
_JPW = _BH // _NW

_mesh = plsc.VectorSubcoreMesh(core_axis_name="c", subcore_axis_name="s")


@functools.partial(
    pl.kernel,
    mesh=_mesh,
    compiler_params=pltpu.CompilerParams(
        use_tc_tiling_on_sc=False, needs_layout_passes=False
    ),
    out_type=jax.ShapeDtypeStruct(
        (HIST_LEN, 8, _BH, 8, 128), jnp.float32
    ),
    scratch_types=[
        pltpu.VMEM((HIST_LEN, 128 * _JPW), jnp.int32),
        pltpu.VMEM((128, EMBEDDING_DIM), jnp.float32),
        pltpu.VMEM((128, EMBEDDING_DIM), jnp.float32),
        pltpu.VMEM((128, EMBEDDING_DIM), jnp.float32),
        pltpu.VMEM((128, EMBEDDING_DIM), jnp.float32),
        pltpu.VMEM((8, 8, 129), jnp.float32),
        pltpu.VMEM((8, 8, 129), jnp.float32),
        pltpu.VMEM((8, 8, 129), jnp.float32),
        pltpu.VMEM((8, 8, 129), jnp.float32),
        pltpu.SemaphoreType.DMA,
        pltpu.SemaphoreType.DMA,
        pltpu.SemaphoreType.DMA,
        pltpu.SemaphoreType.DMA,
        pltpu.SemaphoreType.DMA,
        pltpu.SemaphoreType.DMA,
        pltpu.SemaphoreType.DMA,
        pltpu.SemaphoreType.DMA,
    ],
)
def _gather_kernel(idxt_hbm, table_hbm, out_hbm, idx_v,
                   r0, r1, r2, r3, tb0, tb1, tb2, tb3,
                   g0, g1, g2, g3, s0, s1, s2, s3):
    wid = lax.axis_index("s") * _NC + lax.axis_index("c")
    rows = (r0, r1, r2, r3)
    tbuf = (tb0, tb1, tb2, tb3)
    gsem = (g0, g1, g2, g3)
    ssem = (s0, s1, s2, s3)

    pltpu.sync_copy(idxt_hbm.at[:, pl.ds(wid * 128 * _JPW, 128 * _JPW)], idx_v)

    iota = lax.iota(jnp.int32, 16)
    cdh = [jax.lax.shift_right_logical(iota + 16 * g, 3) for g in range(4)]
    cdl = [jax.lax.bitwise_and(iota + 16 * g, 7) for g in range(4)]
    zvec = jnp.zeros((16,), jnp.int32)

    def gather_desc(t, j):
        return pltpu.make_async_copy(
            table_hbm.at[idx_v.at[t, pl.ds(j * 128, 128)]],
            rows[j],
            gsem[j],
        )

    def store_desc(t, j):
        return pltpu.make_async_copy(
            tbuf[j].at[:, :, pl.ds(0, 128)],
            out_hbm.at[t, :, wid * _JPW + j],
            ssem[j],
        )

    for j in range(_JPW):
        gather_desc(0, j).start()
        store_desc(0, j).start()

    def body(t, carry):
        for j in range(_JPW):
            gather_desc(t, j).wait()
            store_desc(t, j).wait()
            for bl in range(128):
                blvec = jnp.full((16,), t * 0 + bl, dtype=jnp.int32)
                vs = [rows[j][bl, pl.ds(16 * g, 16)] for g in range(4)]
                for g in range(4):
                    plsc.store_scatter(
                        tbuf[j],
                        [cdh[g], cdl[g], blvec],
                        vs[g],
                    )
            @pl.when(t < HIST_LEN - 1)
            def _():
                gather_desc(t + 1, j).start()
            store_desc(t, j).start()
        return carry

    lax.fori_loop(0, HIST_LEN, body, 0)

    for j in range(_JPW):
        store_desc(HIST_LEN - 1, j).wait()


def kernel(indices, table):
    idxt = indices.T.astype(jnp.int32)
    out5 = _gather_kernel(idxt, table)
    return out5.transpose(2, 4, 0, 1, 3).reshape(BATCH, HIST_LEN, EMBEDDING_DIM)


# speedup vs baseline: 4.4038x; 1.7173x over previous
import functools

import jax
import jax.numpy as jnp
from jax import lax
from jax.experimental import pallas as pl
from jax.experimental.pallas import tpu as pltpu
from jax.experimental.pallas import tpu_sc as plsc

TOKENS_DIM = 100000
EMBEDDING_DIM = 64
BATCH = 16384
HIST_LEN = 50
_B = BATCH * HIST_LEN

_info = plsc.get_sparse_core_info()
_NC = _info.num_cores
_NS = _info.num_subcores
_NW = _NC * _NS
_BH = BATCH // 128
_JPW = _BH // _NW

_mesh = plsc.VectorSubcoreMesh(core_axis_name="c", subcore_axis_name="s")


@functools.partial(
    pl.kernel,
    mesh=_mesh,
    compiler_params=pltpu.CompilerParams(
        use_tc_tiling_on_sc=False, needs_layout_passes=False
    ),
    out_type=jax.ShapeDtypeStruct(
        (HIST_LEN, 8, _BH, 8, 128), jnp.float32
    ),
    scratch_types=[
        pltpu.VMEM((HIST_LEN, 128 * _JPW), jnp.int32),
        pltpu.VMEM((128, EMBEDDING_DIM), jnp.float32),
        pltpu.VMEM((128, EMBEDDING_DIM), jnp.float32),
        pltpu.VMEM((128, EMBEDDING_DIM), jnp.float32),
        pltpu.VMEM((128, EMBEDDING_DIM), jnp.float32),
        pltpu.VMEM((8, 8, 129), jnp.float32),
        pltpu.VMEM((8, 8, 129), jnp.float32),
        pltpu.VMEM((8, 8, 129), jnp.float32),
        pltpu.VMEM((8, 8, 129), jnp.float32),
        pltpu.SemaphoreType.DMA,
        pltpu.SemaphoreType.DMA,
        pltpu.SemaphoreType.DMA,
        pltpu.SemaphoreType.DMA,
        pltpu.SemaphoreType.DMA,
        pltpu.SemaphoreType.DMA,
        pltpu.SemaphoreType.DMA,
        pltpu.SemaphoreType.DMA,
    ],
)
def _gather_kernel(idxt_hbm, table_hbm, out_hbm, idx_v,
                   r0, r1, r2, r3, tb0, tb1, tb2, tb3,
                   g0, g1, g2, g3, s0, s1, s2, s3):
    wid = lax.axis_index("s") * _NC + lax.axis_index("c")
    rows = (r0, r1, r2, r3)
    tbuf = (tb0, tb1, tb2, tb3)
    gsem = (g0, g1, g2, g3)
    ssem = (s0, s1, s2, s3)

    pltpu.sync_copy(idxt_hbm.at[:, pl.ds(wid * 128 * _JPW, 128 * _JPW)], idx_v)

    iota = lax.iota(jnp.int32, 16)
    cdh = [jax.lax.shift_right_logical(iota + 16 * g, 3) for g in range(4)]
    cdl = [jax.lax.bitwise_and(iota + 16 * g, 7) for g in range(4)]
    zvec = jnp.zeros((16,), jnp.int32)

    def gather_desc(t, j):
        return pltpu.make_async_copy(
            table_hbm.at[idx_v.at[t, pl.ds(j * 128, 128)]],
            rows[j],
            gsem[j],
        )

    def store_desc(t, j):
        return pltpu.make_async_copy(
            tbuf[j].at[:, :, pl.ds(0, 128)],
            out_hbm.at[t, :, wid * _JPW + j],
            ssem[j],
        )

    for j in range(_JPW):
        gather_desc(0, j).start()
        store_desc(0, j).start()

    def body(t, carry):
        for j in range(_JPW):
            gather_desc(t, j).wait()
            store_desc(t, j).wait()
            def tloop(q, c):
                for u in range(4):
                    bl = 4 * q + u
                    blvec = jnp.full((16,), bl, dtype=jnp.int32)
                    vs = [rows[j][bl, pl.ds(16 * g, 16)] for g in range(4)]
                    for g in range(4):
                        plsc.store_scatter(
                            tbuf[j],
                            [cdh[g], cdl[g], blvec],
                            vs[g],
                        )
                return c

            lax.fori_loop(0, 32, tloop, 0)
            @pl.when(t < HIST_LEN - 1)
            def _():
                gather_desc(t + 1, j).start()
            store_desc(t, j).start()
        return carry

    lax.fori_loop(0, HIST_LEN, body, 0)

    for j in range(_JPW):
        store_desc(HIST_LEN - 1, j).wait()


def kernel(indices, table):
    idxt = indices.T.astype(jnp.int32)
    out5 = _gather_kernel(idxt, table)
    return out5.transpose(2, 4, 0, 1, 3).reshape(BATCH, HIST_LEN, EMBEDDING_DIM)


# speedup vs baseline: 5.5853x; 1.2683x over previous
import functools

import jax
import jax.numpy as jnp
from jax import lax
from jax.experimental import pallas as pl
from jax.experimental.pallas import tpu as pltpu
from jax.experimental.pallas import tpu_sc as plsc

TOKENS_DIM = 100000
EMBEDDING_DIM = 64
BATCH = 16384
HIST_LEN = 50
_B = BATCH * HIST_LEN

_info = plsc.get_sparse_core_info()
_NC = _info.num_cores
_NS = _info.num_subcores
_NW = _NC * _NS
_BH = BATCH // 128
_JPW = _BH // _NW

_mesh = plsc.VectorSubcoreMesh(core_axis_name="c", subcore_axis_name="s")


@functools.partial(
    pl.kernel,
    mesh=_mesh,
    compiler_params=pltpu.CompilerParams(
        use_tc_tiling_on_sc=False, needs_layout_passes=False
    ),
    out_type=jax.ShapeDtypeStruct(
        (HIST_LEN, 8, _BH, 8, 128), jnp.float32
    ),
    scratch_types=[
        pltpu.VMEM((HIST_LEN, 128 * _JPW), jnp.int32),
        pltpu.VMEM((128, EMBEDDING_DIM), jnp.float32),
        pltpu.VMEM((128, EMBEDDING_DIM), jnp.float32),
        pltpu.VMEM((128, EMBEDDING_DIM), jnp.float32),
        pltpu.VMEM((128, EMBEDDING_DIM), jnp.float32),
        pltpu.VMEM((8, 8, 129), jnp.float32),
        pltpu.VMEM((8, 8, 129), jnp.float32),
        pltpu.VMEM((8, 8, 129), jnp.float32),
        pltpu.VMEM((8, 8, 129), jnp.float32),
        pltpu.SemaphoreType.DMA,
        pltpu.SemaphoreType.DMA,
        pltpu.SemaphoreType.DMA,
        pltpu.SemaphoreType.DMA,
        pltpu.SemaphoreType.DMA,
        pltpu.SemaphoreType.DMA,
        pltpu.SemaphoreType.DMA,
        pltpu.SemaphoreType.DMA,
    ],
)
def _gather_kernel(idxt_hbm, table_hbm, out_hbm, idx_v,
                   r0, r1, r2, r3, tb0, tb1, tb2, tb3,
                   g0, g1, g2, g3, s0, s1, s2, s3):
    wid = lax.axis_index("s") * _NC + lax.axis_index("c")
    rows = (r0, r1, r2, r3)
    tbuf = (tb0, tb1, tb2, tb3)
    gsem = (g0, g1, g2, g3)
    ssem = (s0, s1, s2, s3)

    pltpu.sync_copy(idxt_hbm.at[:, pl.ds(wid * 128 * _JPW, 128 * _JPW)], idx_v)

    iota = lax.iota(jnp.int32, 16)
    cdh = [jax.lax.shift_right_logical(iota + 16 * g, 3) for g in range(4)]
    cdl = [jax.lax.bitwise_and(iota + 16 * g, 7) for g in range(4)]
    zvec = jnp.zeros((16,), jnp.int32)

    def gather_desc(t, j):
        return pltpu.make_async_copy(
            table_hbm.at[idx_v.at[t, pl.ds(j * 128, 128)]],
            rows[j],
            gsem[j],
        )

    def store_desc(t, j):
        return pltpu.make_async_copy(
            tbuf[j].at[:, :, pl.ds(0, 128)],
            out_hbm.at[t, :, wid * _JPW + j],
            ssem[j],
        )

    for j in range(_JPW):
        gather_desc(0, j).start()
        store_desc(0, j).start()

    def body(t, carry):
        for j in range(_JPW):
            gather_desc(t, j).wait()
            store_desc(t, j).wait()
            @plsc.parallel_loop(0, 128, unroll=8)
            def _(bl):
                blvec = jnp.full((16,), bl, dtype=jnp.int32)
                vs = [rows[j][bl, pl.ds(16 * g, 16)] for g in range(4)]
                for g in range(4):
                    plsc.store_scatter(
                        tbuf[j],
                        [cdh[g], cdl[g], blvec],
                        vs[g],
                    )
            @pl.when(t < HIST_LEN - 1)
            def _():
                gather_desc(t + 1, j).start()
            store_desc(t, j).start()
        return carry

    lax.fori_loop(0, HIST_LEN, body, 0)

    for j in range(_JPW):
        store_desc(HIST_LEN - 1, j).wait()


def kernel(indices, table):
    idxt = indices.T.astype(jnp.int32)
    out5 = _gather_kernel(idxt, table)
    return out5.transpose(2, 4, 0, 1, 3).reshape(BATCH, HIST_LEN, EMBEDDING_DIM)
